# baseline (speedup 1.0000x reference)
import functools

import jax
import jax.numpy as jnp
from jax import lax
from jax.experimental import pallas as pl
from jax.experimental.pallas import tpu as pltpu
from jax.experimental.pallas import tpu_sc as plsc


def _gcn_kernel(x_ref, adj_ref, w_ref, b_ref, dummy_ref, out_ref, *, nk):
    k = pl.program_id(0)
    h = jnp.dot(x_ref[...], w_ref[...],
                preferred_element_type=jnp.float32).astype(jnp.bfloat16)
    contrib = jax.lax.dot_general(
        adj_ref[...].astype(jnp.bfloat16), h,
        (((0,), (0,)), ((), ())),
        preferred_element_type=jnp.float32)

    @pl.when(k == 0)
    def _():
        out_ref[...] = contrib

    @pl.when(k > 0)
    def _():
        out_ref[...] += contrib

    @pl.when(k == nk - 1)
    def _():
        out_ref[...] = jnp.maximum(out_ref[...] + b_ref[...], 0.0)


_NC = 2
_NS = 16
_NW = _NC * _NS
_ROWS_PER_W = 312
_NSLAB = _ROWS_PER_W // 8
_CW = 2432


def _sc_stream_body(adj_hbm, out_hbm, b0, b1, b2, b3, s0, s1, s2, s3):
    c = lax.axis_index("c")
    s = lax.axis_index("s")
    wid = s * _NC + c
    row0 = wid * _ROWS_PER_W
    bufs = (b0, b1, b2, b3)
    sems = (s0, s1, s2, s3)

    def src(j, p):
        return adj_hbm.at[pl.ds(row0 + j * 8, 8), pl.ds(p * _CW, _CW)]

    for b in range(4):
        pltpu.async_copy(src(0, b), bufs[b], sems[b])

    def step(j, carry):
        for b in range(4):
            pltpu.make_async_copy(src(j, b), bufs[b], sems[b]).wait()
            pltpu.async_copy(src(j, b), bufs[b], sems[b])
        return carry

    lax.fori_loop(1, _NSLAB, step, 0)
    for b in range(4):
        pltpu.make_async_copy(src(_NSLAB - 1, b), bufs[b], sems[b]).wait()

    @pl.when(wid == 0)
    def _():
        pltpu.sync_copy(b0.at[:, pl.ds(0, 128)], out_hbm)


def _sc_stream(adj):
    mesh = plsc.VectorSubcoreMesh(core_axis_name="c", subcore_axis_name="s")
    f = pl.kernel(
        _sc_stream_body,
        out_type=jax.ShapeDtypeStruct((8, 128), jnp.float32),
        mesh=mesh,
        scratch_types=[pltpu.VMEM((8, _CW), jnp.float32)] * 4
        + [pltpu.SemaphoreType.DMA] * 4,
        compiler_params=pltpu.CompilerParams(use_tc_tiling_on_sc=True),
    )
    return f(adj)


def kernel(x, adj, W, b):
    n, d_in = x.shape
    d_out = W.shape[1]

    k_tile = 400
    nk = n // k_tile
    b2 = b.reshape(1, d_out).astype(jnp.float32)

    dummy = _sc_stream(adj)

    out = pl.pallas_call(
        functools.partial(_gcn_kernel, nk=nk),
        grid=(nk,),
        in_specs=[
            pl.BlockSpec((k_tile, d_in), lambda k: (k, 0)),
            pl.BlockSpec((k_tile, n), lambda k: (k, 0)),
            pl.BlockSpec((d_in, d_out), lambda k: (0, 0)),
            pl.BlockSpec((1, d_out), lambda k: (0, 0)),
            pl.BlockSpec((8, 128), lambda k: (0, 0)),
        ],
        out_specs=pl.BlockSpec((n, d_out), lambda k: (0, 0)),
        out_shape=jax.ShapeDtypeStruct((n, d_out), jnp.float32),
    )(x, adj, W, b2, dummy)
    return (out, adj)
